# Initial kernel scaffold; baseline (speedup 1.0000x reference)
import jax
import jax.numpy as jnp
from jax.experimental import pallas as pl


def kernel(hidden_states, W1, b1, W2, b2, expert_weights):
    raise NotImplementedError("write your pallas kernel here")


# speedup vs baseline: 2.0212x; 2.0212x over previous
import jax
import jax.numpy as jnp
from jax import lax
from jax.experimental import pallas as pl
from jax.experimental.pallas import tpu as pltpu

_H = 768
_E = 16
_TOPK = 2
_CAP = 256
_T = 2048
_TB = 128
_G = _T // _TB
_EC = _E * _CAP


def _router_body(x_ref, w1_ref, b1_ref, w2_ref, b2_ref, ew_ref,
                 disp_ref, comb_ref, probs_ref, aux_ref, acc_ref):
    i = pl.program_id(0)

    @pl.when(i == 0)
    def _init():
        acc_ref[...] = jnp.zeros_like(acc_ref)

    h = jnp.dot(x_ref[...], w1_ref[...], preferred_element_type=jnp.float32)
    h = jnp.maximum(h + b1_ref[...], 0.0)
    logits = jnp.dot(h, w2_ref[...], preferred_element_type=jnp.float32)
    logits = (logits + b2_ref[...]) * ew_ref[...]

    m = jnp.max(logits, axis=1, keepdims=True)
    ex = jnp.exp(logits - m)
    p = ex / jnp.sum(ex, axis=1, keepdims=True)
    probs_ref[...] = p

    iota_e = lax.broadcasted_iota(jnp.int32, (_TB, _E), 1)
    p0 = jnp.max(p, axis=1, keepdims=True)
    i0 = jnp.min(jnp.where(p == p0, iota_e, _E), axis=1, keepdims=True)
    oh0 = iota_e == i0
    pm = jnp.where(oh0, -1.0, p)
    p1 = jnp.max(pm, axis=1, keepdims=True)
    i1 = jnp.min(jnp.where(pm == p1, iota_e, _E), axis=1, keepdims=True)
    oh1 = iota_e == i1
    s = p0 + p1 + 1e-8
    w0 = p0 / s
    w1 = p1 / s

    oh0f = oh0.astype(jnp.float32)
    oh1f = oh1.astype(jnp.float32)
    s2 = oh0f + oh1f
    row = lax.broadcasted_iota(jnp.int32, (_TB, _TB), 0)
    col = lax.broadcasted_iota(jnp.int32, (_TB, _TB), 1)
    stril = (row > col).astype(jnp.float32)
    c_in = jnp.dot(stril, s2, preferred_element_type=jnp.float32)
    carry = acc_ref[0:1, 0:_E]
    c_tot = c_in + carry
    pos0 = jnp.sum(c_tot * oh0f, axis=1, keepdims=True).astype(jnp.int32)
    pos1 = jnp.sum(c_tot * oh1f, axis=1, keepdims=True).astype(jnp.int32)
    acc_ref[0:1, 0:_E] = carry + jnp.sum(s2, axis=0, keepdims=True)
    acc_ref[1:2, 0:_E] += jnp.sum(p, axis=0, keepdims=True)

    q0 = jnp.where(pos0 < _CAP, i0 * _CAP + pos0, -1)
    q1 = jnp.where(pos1 < _CAP, i1 * _CAP + pos1, -1)
    qq = lax.broadcasted_iota(jnp.int32, (_TB, _EC), 1)
    m0 = qq == q0
    m1 = qq == q1
    disp_ref[...] = m0.astype(jnp.float32) + m1.astype(jnp.float32)
    comb_ref[...] = jnp.where(m0, w0, 0.0) + jnp.where(m1, w1, 0.0)

    cnt = acc_ref[0:1, 0:_E]
    psum = acc_ref[1:2, 0:_E]
    aux_ref[...] = (0.1 * _E) * jnp.sum(
        (psum / _T) * (cnt / (_T * _TOPK)), axis=1, keepdims=True)


def kernel(hidden_states, W1, b1, W2, b2, expert_weights):
    Bv, Sv, Hv = hidden_states.shape
    x = hidden_states.reshape(Bv * Sv, Hv)
    w1t = W1.T
    w2t = W2.T
    b1r = b1.reshape(1, Hv)
    b2r = b2.reshape(1, _E)
    ewr = expert_weights.reshape(1, _E)

    disp, comb, probs, aux = pl.pallas_call(
        _router_body,
        grid=(_G,),
        in_specs=[
            pl.BlockSpec((_TB, _H), lambda i: (i, 0)),
            pl.BlockSpec((_H, _H), lambda i: (0, 0)),
            pl.BlockSpec((1, _H), lambda i: (0, 0)),
            pl.BlockSpec((_H, _E), lambda i: (0, 0)),
            pl.BlockSpec((1, _E), lambda i: (0, 0)),
            pl.BlockSpec((1, _E), lambda i: (0, 0)),
        ],
        out_specs=[
            pl.BlockSpec((_TB, _EC), lambda i: (i, 0)),
            pl.BlockSpec((_TB, _EC), lambda i: (i, 0)),
            pl.BlockSpec((_TB, _E), lambda i: (i, 0)),
            pl.BlockSpec((1, 1), lambda i: (0, 0)),
        ],
        out_shape=[
            jax.ShapeDtypeStruct((_T, _EC), jnp.float32),
            jax.ShapeDtypeStruct((_T, _EC), jnp.float32),
            jax.ShapeDtypeStruct((_T, _E), jnp.float32),
            jax.ShapeDtypeStruct((1, 1), jnp.float32),
        ],
        scratch_shapes=[pltpu.VMEM((8, 128), jnp.float32)],
    )(x, w1t, b1r, w2t, b2r, ewr)

    dispatch = disp.reshape(Bv, Sv, _E, _CAP)
    combine = comb.reshape(Bv, Sv, _E, _CAP)
    router_probs = probs.reshape(Bv, Sv, _E)
    return dispatch, combine, router_probs, aux.reshape(())


# speedup vs baseline: 2.1901x; 1.0836x over previous
import jax
import jax.numpy as jnp
from jax import lax
from jax.experimental import pallas as pl
from jax.experimental.pallas import tpu as pltpu

_H = 768
_E = 16
_TOPK = 2
_CAP = 256
_T = 2048
_TB = 256
_G = _T // _TB
_EC = _E * _CAP


def _router_body(x_ref, w1_ref, b1_ref, w2_ref, b2_ref, ew_ref,
                 disp_ref, comb_ref, probs_ref, aux_ref, acc_ref):
    i = pl.program_id(0)

    @pl.when(i == 0)
    def _init():
        acc_ref[...] = jnp.zeros_like(acc_ref)

    h = jnp.dot(x_ref[...], w1_ref[...], preferred_element_type=jnp.float32)
    h = jnp.maximum(h + b1_ref[...], 0.0)
    logits = jnp.dot(h, w2_ref[...], preferred_element_type=jnp.float32)
    logits = (logits + b2_ref[...]) * ew_ref[...]

    m = jnp.max(logits, axis=1, keepdims=True)
    ex = jnp.exp(logits - m)
    p = ex / jnp.sum(ex, axis=1, keepdims=True)
    probs_ref[...] = p

    iota_e = lax.broadcasted_iota(jnp.int32, (_TB, _E), 1)
    p0 = jnp.max(p, axis=1, keepdims=True)
    i0 = jnp.min(jnp.where(p == p0, iota_e, _E), axis=1, keepdims=True)
    oh0 = iota_e == i0
    pm = jnp.where(oh0, -1.0, p)
    p1 = jnp.max(pm, axis=1, keepdims=True)
    i1 = jnp.min(jnp.where(pm == p1, iota_e, _E), axis=1, keepdims=True)
    oh1 = iota_e == i1
    s = p0 + p1 + 1e-8
    w0 = p0 / s
    w1 = p1 / s

    oh0f = oh0.astype(jnp.float32)
    oh1f = oh1.astype(jnp.float32)
    s2 = oh0f + oh1f
    row = lax.broadcasted_iota(jnp.int32, (_TB, _TB), 0)
    col = lax.broadcasted_iota(jnp.int32, (_TB, _TB), 1)
    stril = (row > col).astype(jnp.float32)
    c_in = jnp.dot(stril, s2, preferred_element_type=jnp.float32)
    carry = acc_ref[0:1, 0:_E]
    c_tot = c_in + carry
    pos0 = jnp.sum(c_tot * oh0f, axis=1, keepdims=True).astype(jnp.int32)
    pos1 = jnp.sum(c_tot * oh1f, axis=1, keepdims=True).astype(jnp.int32)
    acc_ref[0:1, 0:_E] = carry + jnp.sum(s2, axis=0, keepdims=True)
    acc_ref[1:2, 0:_E] += jnp.sum(p, axis=0, keepdims=True)

    q0 = jnp.where(pos0 < _CAP, i0 * _CAP + pos0, -1)
    q1 = jnp.where(pos1 < _CAP, i1 * _CAP + pos1, -1)
    qq = lax.broadcasted_iota(jnp.int32, (_TB, _EC), 1)
    m0 = qq == q0
    m1 = qq == q1
    disp_ref[...] = m0.astype(jnp.float32) + m1.astype(jnp.float32)
    comb_ref[...] = jnp.where(m0, w0, 0.0) + jnp.where(m1, w1, 0.0)

    cnt = acc_ref[0:1, 0:_E]
    psum = acc_ref[1:2, 0:_E]
    aux_ref[...] = (0.1 * _E) * jnp.sum(
        (psum / _T) * (cnt / (_T * _TOPK)), axis=1, keepdims=True)


def kernel(hidden_states, W1, b1, W2, b2, expert_weights):
    Bv, Sv, Hv = hidden_states.shape
    x = hidden_states.reshape(Bv * Sv, Hv)
    w1t = W1.T
    w2t = W2.T
    b1r = b1.reshape(1, Hv)
    b2r = b2.reshape(1, _E)
    ewr = expert_weights.reshape(1, _E)

    disp, comb, probs, aux = pl.pallas_call(
        _router_body,
        grid=(_G,),
        in_specs=[
            pl.BlockSpec((_TB, _H), lambda i: (i, 0)),
            pl.BlockSpec((_H, _H), lambda i: (0, 0)),
            pl.BlockSpec((1, _H), lambda i: (0, 0)),
            pl.BlockSpec((_H, _E), lambda i: (0, 0)),
            pl.BlockSpec((1, _E), lambda i: (0, 0)),
            pl.BlockSpec((1, _E), lambda i: (0, 0)),
        ],
        out_specs=[
            pl.BlockSpec((_TB, _EC), lambda i: (i, 0)),
            pl.BlockSpec((_TB, _EC), lambda i: (i, 0)),
            pl.BlockSpec((_TB, _E), lambda i: (i, 0)),
            pl.BlockSpec((1, 1), lambda i: (0, 0)),
        ],
        out_shape=[
            jax.ShapeDtypeStruct((_T, _EC), jnp.float32),
            jax.ShapeDtypeStruct((_T, _EC), jnp.float32),
            jax.ShapeDtypeStruct((_T, _E), jnp.float32),
            jax.ShapeDtypeStruct((1, 1), jnp.float32),
        ],
        scratch_shapes=[pltpu.VMEM((8, 128), jnp.float32)],
    )(x, w1t, b1r, w2t, b2r, ewr)

    dispatch = disp.reshape(Bv, Sv, _E, _CAP)
    combine = comb.reshape(Bv, Sv, _E, _CAP)
    router_probs = probs.reshape(Bv, Sv, _E)
    return dispatch, combine, router_probs, aux.reshape(())
